# baseline (speedup 1.0000x reference)
import functools

import jax
import jax.numpy as jnp
from jax import lax
from jax.experimental import pallas as pl
from jax.experimental.pallas import tpu as pltpu
from jax.experimental.pallas import tpu_sc as plsc

_VOCAB = 500000
_DIM = 64
_PRIME = 31

_B = 8
_T = 512
_NW = 32
_CHUNK = (_B * _T) // _NW
_CHUNKS_PER_ROW = _T // _CHUNK
_PAD = 8
_L = 16


def _body(byte_hbm, emb2_hbm, emb3_hbm, emb4_hbm, out_hbm,
          bytes_v, idx2_v, idx3_v, idx4_v, off2_v, off3_v, off4_v,
          g2_v, g3_v, g4_v, o_v, sem):
    nc = 2
    wid = lax.axis_index("s") * nc + lax.axis_index("c")
    b = wid // _CHUNKS_PER_ROW
    p0 = (wid % _CHUNKS_PER_ROW) * _CHUNK

    bytes_v[pl.ds(0, _L)] = jnp.zeros((_L,), jnp.int32)
    pltpu.sync_copy(byte_hbm, bytes_v.at[pl.ds(_PAD, _B * _T)])

    base = b * _T + p0
    for g in range(_CHUNK // _L):
        t0 = base + g * _L
        v0 = bytes_v[pl.ds(_PAD + t0, _L)]
        v1 = bytes_v[pl.ds(_PAD + t0 - 1, _L)]
        v2 = bytes_v[pl.ds(_PAD + t0 - 2, _L)]
        v3 = bytes_v[pl.ds(_PAD + t0 - 3, _L)]
        h2 = v1 * _PRIME + v0
        h3 = v2 * (_PRIME * _PRIME) + h2
        h4 = lax.rem(v3 * (_PRIME * _PRIME * _PRIME) + h3, _VOCAB)
        sl = pl.ds(g * _L, _L)
        idx2_v[sl] = lax.shift_right_logical(h2, 1)
        idx3_v[sl] = lax.shift_right_logical(h3, 1)
        idx4_v[sl] = lax.shift_right_logical(h4, 1)
        off2_v[sl] = (h2 & 1) * _DIM
        off3_v[sl] = (h3 & 1) * _DIM
        off4_v[sl] = (h4 & 1) * _DIM

    c2 = pltpu.async_copy(emb2_hbm.at[idx2_v], g2_v, sem)
    c3 = pltpu.async_copy(emb3_hbm.at[idx3_v], g3_v, sem)
    c4 = pltpu.async_copy(emb4_hbm.at[idx4_v], g4_v, sem)
    c2.wait()
    c3.wait()
    c4.wait()

    def acc_body(t, _):
        s2 = off2_v[pl.ds(t, _L)][0]
        s3 = off3_v[pl.ds(t, _L)][0]
        s4 = off4_v[pl.ds(t, _L)][0]
        for c in range(_DIM // _L):
            v2 = g2_v[t, pl.ds(s2 + c * _L, _L)]
            v3 = g3_v[t, pl.ds(s3 + c * _L, _L)]
            v4 = g4_v[t, pl.ds(s4 + c * _L, _L)]
            o_v[t, pl.ds(c * _L, _L)] = (v2 + v3 + v4) * 0.25
        return 0

    lax.fori_loop(0, _CHUNK, acc_body, 0)

    @pl.when(p0 == 0)
    def _fixup():
        s2a = off2_v[pl.ds(1, _L)][0]
        s2b = off2_v[pl.ds(2, _L)][0]
        s3b = off3_v[pl.ds(2, _L)][0]
        for c in range(_DIM // _L):
            sl = pl.ds(c * _L, _L)
            o_v[0, sl] = jnp.zeros((_L,), jnp.float32)
            o_v[1, sl] = g2_v[1, pl.ds(s2a + c * _L, _L)] * 0.25
            o_v[2, sl] = (g2_v[2, pl.ds(s2b + c * _L, _L)]
                          + g3_v[2, pl.ds(s3b + c * _L, _L)]) * 0.25

    pltpu.sync_copy(o_v, out_hbm.at[b, pl.ds(p0, _CHUNK), :])


@jax.jit
def kernel(byte_ids, emb_2, emb_3, emb_4):
    mesh = plsc.VectorSubcoreMesh(core_axis_name="c", subcore_axis_name="s")
    f = functools.partial(
        pl.kernel,
        mesh=mesh,
        compiler_params=pltpu.CompilerParams(use_tc_tiling_on_sc=True),
        out_type=jax.ShapeDtypeStruct((_B, _T, 2 * _DIM), jnp.float32),
        scratch_types=[
            pltpu.VMEM((_PAD + _B * _T,), jnp.int32),
            pltpu.VMEM((_CHUNK,), jnp.int32),
            pltpu.VMEM((_CHUNK,), jnp.int32),
            pltpu.VMEM((_CHUNK,), jnp.int32),
            pltpu.VMEM((_CHUNK + _L,), jnp.int32),
            pltpu.VMEM((_CHUNK + _L,), jnp.int32),
            pltpu.VMEM((_CHUNK + _L,), jnp.int32),
            pltpu.VMEM((_CHUNK, 2 * _DIM), jnp.float32),
            pltpu.VMEM((_CHUNK, 2 * _DIM), jnp.float32),
            pltpu.VMEM((_CHUNK, 2 * _DIM), jnp.float32),
            pltpu.VMEM((_CHUNK, 2 * _DIM), jnp.float32),
            pltpu.SemaphoreType.DMA,
        ],
    )(_body)
    out_pad = f(
        byte_ids.reshape(-1),
        emb_2.reshape(_VOCAB // 2, 2 * _DIM),
        emb_3.reshape(_VOCAB // 2, 2 * _DIM),
        emb_4.reshape(_VOCAB // 2, 2 * _DIM),
    )
    return out_pad[:, :, :_DIM]
